# baseline (device time: 15943 ns/iter reference)
import jax
import jax.numpy as jnp
from jax import lax
from jax.experimental import pallas as pl
from jax.experimental.pallas import tpu as pltpu

CHUNK_ROWS = (64, 56, 48, 40, 24, 16, 8)
CHUNKS = len(CHUNK_ROWS)
CHUNK_OFFS = tuple(sum(CHUNK_ROWS[:c]) for c in range(CHUNKS))


def kernel(x):
    m_per, n = x.shape
    m_half = m_per // 2
    assert sum(CHUNK_ROWS) == m_half

    def body(x_ref, out_ref, s1, r1, s2, r2, local_sem):
        my_x = lax.axis_index("x")
        my_y = lax.axis_index("y")
        other_x = 1 - my_x
        other_y = 1 - my_y

        local_copy = pltpu.make_async_copy(
            x_ref, out_ref.at[pl.ds(my_x * m_per, m_per), :], local_sem
        )
        local_copy.start()

        barrier_sem = pltpu.get_barrier_semaphore()
        pl.semaphore_signal(
            barrier_sem, inc=1,
            device_id=(other_x, my_y), device_id_type=pl.DeviceIdType.MESH,
        )
        pl.semaphore_signal(
            barrier_sem, inc=1,
            device_id=(my_x, other_y), device_id_type=pl.DeviceIdType.MESH,
        )
        pl.semaphore_wait(barrier_sem, 2)

        p1_dst_base = my_x * m_per + my_y * m_half
        p1_rcv_base = other_x * m_per + my_y * m_half

        p1 = []
        for c in range(CHUNKS):
            off, rows = CHUNK_OFFS[c], CHUNK_ROWS[c]
            rdma = pltpu.make_async_remote_copy(
                src_ref=x_ref.at[pl.ds(my_y * m_half + off, rows), :],
                dst_ref=out_ref.at[pl.ds(p1_dst_base + off, rows), :],
                send_sem=s1.at[c],
                recv_sem=r1.at[c],
                device_id=(other_x, my_y),
                device_id_type=pl.DeviceIdType.MESH,
            )
            rdma.start()
            p1.append(rdma)

        p2 = []
        for c in range(CHUNKS):
            p1[c].wait_recv()
            rows = pl.ds(p1_rcv_base + CHUNK_OFFS[c], CHUNK_ROWS[c])
            rdma = pltpu.make_async_remote_copy(
                src_ref=out_ref.at[rows, :],
                dst_ref=out_ref.at[rows, :],
                send_sem=s2.at[c],
                recv_sem=r2.at[c],
                device_id=(my_x, other_y),
                device_id_type=pl.DeviceIdType.MESH,
            )
            rdma.start()
            p2.append(rdma)

        for c in range(CHUNKS):
            p1[c].wait_send()
            p2[c].wait_send()
            p2[c].wait_recv()
        local_copy.wait()

    return pl.pallas_call(
        body,
        out_shape=jax.ShapeDtypeStruct((2 * m_per, n), x.dtype),
        in_specs=[pl.BlockSpec(memory_space=pltpu.VMEM)],
        out_specs=pl.BlockSpec(memory_space=pltpu.VMEM),
        scratch_shapes=[
            pltpu.SemaphoreType.DMA((CHUNKS,)),
            pltpu.SemaphoreType.DMA((CHUNKS,)),
            pltpu.SemaphoreType.DMA((CHUNKS,)),
            pltpu.SemaphoreType.DMA((CHUNKS,)),
            pltpu.SemaphoreType.DMA,
        ],
        compiler_params=pltpu.CompilerParams(collective_id=0),
    )(x)


# device time: 15175 ns/iter; 1.0506x vs baseline; 1.0506x over previous
import jax
import jax.numpy as jnp
from jax import lax
from jax.experimental import pallas as pl
from jax.experimental.pallas import tpu as pltpu

CHUNK_ROWS = (16,) * 16
CHUNKS = len(CHUNK_ROWS)
CHUNK_OFFS = tuple(sum(CHUNK_ROWS[:c]) for c in range(CHUNKS))


def kernel(x):
    m_per, n = x.shape
    m_half = m_per // 2
    assert sum(CHUNK_ROWS) == m_half

    def body(x_ref, out_ref, s1, r1, s2, r2, local_sem):
        my_x = lax.axis_index("x")
        my_y = lax.axis_index("y")
        other_x = 1 - my_x
        other_y = 1 - my_y

        local_copy = pltpu.make_async_copy(
            x_ref, out_ref.at[pl.ds(my_x * m_per, m_per), :], local_sem
        )
        local_copy.start()

        barrier_sem = pltpu.get_barrier_semaphore()
        pl.semaphore_signal(
            barrier_sem, inc=1,
            device_id=(other_x, my_y), device_id_type=pl.DeviceIdType.MESH,
        )
        pl.semaphore_signal(
            barrier_sem, inc=1,
            device_id=(my_x, other_y), device_id_type=pl.DeviceIdType.MESH,
        )
        pl.semaphore_wait(barrier_sem, 2)

        p1_dst_base = my_x * m_per + my_y * m_half
        p1_rcv_base = other_x * m_per + my_y * m_half

        p1 = []
        for c in range(CHUNKS):
            off, rows = CHUNK_OFFS[c], CHUNK_ROWS[c]
            rdma = pltpu.make_async_remote_copy(
                src_ref=x_ref.at[pl.ds(my_y * m_half + off, rows), :],
                dst_ref=out_ref.at[pl.ds(p1_dst_base + off, rows), :],
                send_sem=s1.at[c],
                recv_sem=r1.at[c],
                device_id=(other_x, my_y),
                device_id_type=pl.DeviceIdType.MESH,
            )
            rdma.start()
            p1.append(rdma)

        p2 = []
        for c in range(CHUNKS):
            p1[c].wait_recv()
            rows = pl.ds(p1_rcv_base + CHUNK_OFFS[c], CHUNK_ROWS[c])
            rdma = pltpu.make_async_remote_copy(
                src_ref=out_ref.at[rows, :],
                dst_ref=out_ref.at[rows, :],
                send_sem=s2.at[c],
                recv_sem=r2.at[c],
                device_id=(my_x, other_y),
                device_id_type=pl.DeviceIdType.MESH,
            )
            rdma.start()
            p2.append(rdma)

        for c in range(CHUNKS):
            p1[c].wait_send()
            p2[c].wait_send()
            p2[c].wait_recv()
        local_copy.wait()

    return pl.pallas_call(
        body,
        out_shape=jax.ShapeDtypeStruct((2 * m_per, n), x.dtype),
        in_specs=[pl.BlockSpec(memory_space=pltpu.VMEM)],
        out_specs=pl.BlockSpec(memory_space=pltpu.VMEM),
        scratch_shapes=[
            pltpu.SemaphoreType.DMA((CHUNKS,)),
            pltpu.SemaphoreType.DMA((CHUNKS,)),
            pltpu.SemaphoreType.DMA((CHUNKS,)),
            pltpu.SemaphoreType.DMA((CHUNKS,)),
            pltpu.SemaphoreType.DMA,
        ],
        compiler_params=pltpu.CompilerParams(collective_id=0),
    )(x)
